# Initial kernel scaffold; baseline (speedup 1.0000x reference)
import jax
import jax.numpy as jnp
from jax.experimental import pallas as pl


def kernel(x, edge_index):
    raise NotImplementedError("write your pallas kernel here")


# speedup vs baseline: 5.7938x; 5.7938x over previous
import functools

import jax
import jax.numpy as jnp
from jax import lax
from jax.experimental import pallas as pl
from jax.experimental.pallas import tpu as pltpu
from jax.experimental.pallas import tpu_sc as plsc

N_NODES = 10000
N_EDGES = 320000
D_FEAT = 128
W_PAD = 144
NC, NS = 2, 16
NW = NC * NS
E_PER_TILE = N_EDGES // NW
CHUNK = 80
N_CHUNKS = E_PER_TILE // CHUNK
ROWS_PER_TILE = N_NODES // NS


def _sc_body(x_hbm, ei_hbm, zeros_hbm, out0_hbm, out1_hbm,
             src_v, dst_v, rows_v, acc_sh, sem):
    c = lax.axis_index("c")
    s = lax.axis_index("s")
    wid = c * NS + s

    row0 = pl.multiple_of(s * ROWS_PER_TILE, 8)
    pltpu.sync_copy(zeros_hbm, acc_sh.at[pl.ds(row0, ROWS_PER_TILE)])
    plsc.subcore_barrier()

    base = wid * E_PER_TILE

    def body(i, carry):
        off = pl.multiple_of(base + i * CHUNK, 8)
        pltpu.sync_copy(ei_hbm.at[0, pl.ds(off, CHUNK)], src_v)
        pltpu.sync_copy(ei_hbm.at[1, pl.ds(off, CHUNK)], dst_v)
        pltpu.async_copy(x_hbm.at[src_v], rows_v, sem).wait()
        pltpu.sync_copy(rows_v, acc_sh.at[dst_v], add=True)
        return carry

    lax.fori_loop(0, N_CHUNKS, body, 0)
    plsc.subcore_barrier()

    @pl.when(c == 0)
    def _():
        pltpu.sync_copy(acc_sh.at[pl.ds(row0, ROWS_PER_TILE)],
                        out0_hbm.at[pl.ds(row0, ROWS_PER_TILE)])

    @pl.when(c == 1)
    def _():
        pltpu.sync_copy(acc_sh.at[pl.ds(row0, ROWS_PER_TILE)],
                        out1_hbm.at[pl.ds(row0, ROWS_PER_TILE)])


_sc_call = pl.kernel(
    _sc_body,
    out_type=(
        jax.ShapeDtypeStruct((N_NODES, W_PAD), jnp.float32),
        jax.ShapeDtypeStruct((N_NODES, W_PAD), jnp.float32),
    ),
    mesh=plsc.VectorSubcoreMesh(core_axis_name="c", subcore_axis_name="s"),
    compiler_params=pltpu.CompilerParams(use_tc_tiling_on_sc=False),
    scratch_types=(
        pltpu.VMEM((CHUNK,), jnp.int32),
        pltpu.VMEM((CHUNK,), jnp.int32),
        pltpu.VMEM((CHUNK, W_PAD), jnp.float32),
        pltpu.VMEM_SHARED((N_NODES, W_PAD), jnp.float32),
        pltpu.SemaphoreType.DMA,
    ),
)


def _combine_body(a_ref, b_ref, o_ref):
    s = a_ref[:, :D_FEAT] + b_ref[:, :D_FEAT]
    d = a_ref[:, D_FEAT:D_FEAT + 1] + b_ref[:, D_FEAT:D_FEAT + 1]
    o_ref[:, :] = s / jnp.maximum(d, 1e-8)


_combine = pl.pallas_call(
    _combine_body,
    out_shape=jax.ShapeDtypeStruct((N_NODES, D_FEAT), jnp.float32),
)


@jax.jit
def kernel(x, edge_index):
    pad = jnp.concatenate(
        [jnp.ones((N_NODES, 1), jnp.float32),
         jnp.zeros((N_NODES, W_PAD - D_FEAT - 1), jnp.float32)], axis=1)
    x_pad = jnp.concatenate([x, pad], axis=1)
    zeros = jnp.zeros((ROWS_PER_TILE, W_PAD), jnp.float32)
    p0, p1 = _sc_call(x_pad, edge_index.astype(jnp.int32), zeros)
    return _combine(p0, p1)


# speedup vs baseline: 11.5956x; 2.0014x over previous
import jax
import jax.numpy as jnp
from jax import lax
from jax.experimental import pallas as pl
from jax.experimental.pallas import tpu as pltpu
from jax.experimental.pallas import tpu_sc as plsc

N_NODES = 10000
N_EDGES = 320000
D_FEAT = 128
W_PAD = 144
NC, NS = 2, 16
NW = NC * NS
E_PER_TILE = N_EDGES // NW
CHUNK = 100
N_SEC = 5
SEC_CHUNKS = 20
ROWS_PER_TILE = N_NODES // NS


def _sc_body(x_hbm, ei_hbm, zeros_hbm, out0_hbm, out1_hbm,
             isec0, isec1, rows0_v, rows1_v, acc_sh,
             isem0, isem1, gsem0, gsem1):
    c = lax.axis_index("c")
    s = lax.axis_index("s")
    wid = c * NS + s

    isecs, isems = (isec0, isec1), (isem0, isem1)
    bufs, gsems = (rows0_v, rows1_v), (gsem0, gsem1)

    pltpu.async_copy(ei_hbm.at[wid, 0], isec0, isem0)
    row0 = pl.multiple_of(s * ROWS_PER_TILE, 8)
    pltpu.sync_copy(zeros_hbm, acc_sh.at[pl.ds(row0, ROWS_PER_TILE)])
    plsc.subcore_barrier()

    for sec in range(N_SEC):
        ib = isecs[sec % 2]
        if sec + 1 < N_SEC:
            pltpu.async_copy(ei_hbm.at[wid, sec + 1],
                             isecs[(sec + 1) % 2], isems[(sec + 1) % 2])
        pltpu.make_async_copy(ei_hbm.at[wid, sec], ib, isems[sec % 2]).wait()

        pltpu.async_copy(x_hbm.at[ib.at[0, 0]], rows0_v, gsem0)
        pltpu.async_copy(x_hbm.at[ib.at[1, 0]], rows1_v, gsem1)

        def body(j, carry):
            for b in (0, 1):
                k = 2 * j + b
                pltpu.make_async_copy(x_hbm.at[ib.at[k, 0]],
                                      bufs[b], gsems[b]).wait()
                nxt = k + 2

                @pl.when(nxt < SEC_CHUNKS)
                def _():
                    pltpu.async_copy(x_hbm.at[ib.at[nxt, 0]], bufs[b], gsems[b])

                pltpu.sync_copy(bufs[b], acc_sh.at[ib.at[k, 1]], add=True)
            return carry

        lax.fori_loop(0, SEC_CHUNKS // 2, body, 0)

    plsc.subcore_barrier()

    @pl.when(c == 0)
    def _():
        pltpu.sync_copy(acc_sh.at[pl.ds(row0, ROWS_PER_TILE)],
                        out0_hbm.at[pl.ds(row0, ROWS_PER_TILE)])

    @pl.when(c == 1)
    def _():
        pltpu.sync_copy(acc_sh.at[pl.ds(row0, ROWS_PER_TILE)],
                        out1_hbm.at[pl.ds(row0, ROWS_PER_TILE)])


_sc_call = pl.kernel(
    _sc_body,
    out_type=(
        jax.ShapeDtypeStruct((N_NODES, W_PAD), jnp.float32),
        jax.ShapeDtypeStruct((N_NODES, W_PAD), jnp.float32),
    ),
    mesh=plsc.VectorSubcoreMesh(core_axis_name="c", subcore_axis_name="s"),
    compiler_params=pltpu.CompilerParams(use_tc_tiling_on_sc=False),
    scratch_types=(
        pltpu.VMEM((SEC_CHUNKS, 2, CHUNK), jnp.int32),
        pltpu.VMEM((SEC_CHUNKS, 2, CHUNK), jnp.int32),
        pltpu.VMEM((CHUNK, W_PAD), jnp.float32),
        pltpu.VMEM((CHUNK, W_PAD), jnp.float32),
        pltpu.VMEM_SHARED((N_NODES, W_PAD), jnp.float32),
        pltpu.SemaphoreType.DMA,
        pltpu.SemaphoreType.DMA,
        pltpu.SemaphoreType.DMA,
        pltpu.SemaphoreType.DMA,
    ),
)


def _combine_body(a_ref, b_ref, o_ref):
    s = a_ref[:, :D_FEAT] + b_ref[:, :D_FEAT]
    d = a_ref[:, D_FEAT:D_FEAT + 1] + b_ref[:, D_FEAT:D_FEAT + 1]
    o_ref[:, :] = s / jnp.maximum(d, 1e-8)


_combine = pl.pallas_call(
    _combine_body,
    out_shape=jax.ShapeDtypeStruct((N_NODES, D_FEAT), jnp.float32),
)


@jax.jit
def kernel(x, edge_index):
    pad = jnp.concatenate(
        [jnp.ones((N_NODES, 1), jnp.float32),
         jnp.zeros((N_NODES, W_PAD - D_FEAT - 1), jnp.float32)], axis=1)
    x_pad = jnp.concatenate([x, pad], axis=1)
    zeros = jnp.zeros((ROWS_PER_TILE, W_PAD), jnp.float32)
    ei = edge_index.astype(jnp.int32).reshape(2, NW, N_SEC, SEC_CHUNKS, CHUNK)
    ei = jnp.transpose(ei, (1, 2, 3, 0, 4))
    p0, p1 = _sc_call(x_pad, ei, zeros)
    return _combine(p0, p1)
